# baseline (speedup 1.0000x reference)
import jax
import jax.numpy as jnp
from jax import lax
from jax.experimental import pallas as pl
from jax.experimental.pallas import tpu as pltpu
from jax.experimental.pallas import tpu_sc as plsc

BATCH = 4096
N_TOKENS = 50
TOKENS = BATCH * N_TOKENS
CHUNK = 128
NCHUNKS = TOKENS // CHUNK
NW = 32
CPW = NCHUNKS // NW
D_OUT = 256
N_ROWS = 100000
SEG = 16
RPT = D_OUT // SEG
GLEN = CHUNK * RPT
TABLE_WIDTHS = (64, 32, 16, 16, 32)


def _body(idx_hbm, tab_hbm, out_hbm, idxv, gbuf, gsem0, gsem1, wsem0, wsem1,
          isem):
    gsems = (gsem0, gsem1)
    wsems = (wsem0, wsem1)
    wid = lax.axis_index("s") * 2 + lax.axis_index("c")
    chunk0 = wid * CPW

    def fire_gather(b):
        pltpu.async_copy(tab_hbm.at[idxv.at[b]], gbuf.at[b], gsems[b])

    def wait_gather(b):
        pltpu.make_async_copy(tab_hbm.at[idxv.at[b]], gbuf.at[b],
                              gsems[b]).wait()

    def fire_write(b, g):
        row = pl.multiple_of(g * GLEN, GLEN)
        pltpu.async_copy(gbuf.at[b], out_hbm.at[pl.ds(row, GLEN)], wsems[b])

    def wait_write(b):
        pltpu.make_async_copy(gbuf.at[b], out_hbm.at[pl.ds(0, GLEN)],
                              wsems[b]).wait()

    pltpu.sync_copy(idx_hbm.at[chunk0], idxv.at[0])
    fire_gather(0)

    def pair(jj, carry):
        for b in (0, 1):
            j = jj * 2 + b
            g = chunk0 + j

            @pl.when(j + 1 < CPW)
            def _prefetch_idx():
                pltpu.async_copy(idx_hbm.at[g + 1], idxv.at[1 - b], isem)

            @pl.when(j >= 1)
            def _():
                wait_write(1 - b)

            @pl.when(j + 1 < CPW)
            def _next_gather():
                pltpu.make_async_copy(idx_hbm.at[g + 1], idxv.at[1 - b],
                                      isem).wait()
                fire_gather(1 - b)

            wait_gather(b)
            fire_write(b, g)

        return carry

    lax.fori_loop(0, CPW // 2, pair, 0)
    wait_write((CPW - 1) % 2)


@jax.jit
def _run(idx, tab):
    scratch = [
        pltpu.VMEM((2, GLEN), jnp.int32),
        pltpu.VMEM((2, GLEN, SEG), jnp.float32),
        pltpu.SemaphoreType.DMA,
        pltpu.SemaphoreType.DMA,
        pltpu.SemaphoreType.DMA,
        pltpu.SemaphoreType.DMA,
        pltpu.SemaphoreType.DMA,
    ]
    kern = pl.kernel(
        _body,
        out_type=jax.ShapeDtypeStruct((TOKENS * RPT, SEG), jnp.float32),
        mesh=plsc.VectorSubcoreMesh(core_axis_name="c", subcore_axis_name="s"),
        scratch_types=scratch,
        compiler_params=pltpu.CompilerParams(use_tc_tiling_on_sc=False),
    )
    return kern(idx, tab)


def kernel(int_ids, species_emb, move_emb, ability_emb, item_emb, last_move_emb):
    ids = int_ids.astype(jnp.int32)
    flat = jnp.concatenate([
        species_emb.reshape(-1), move_emb.reshape(-1), ability_emb.reshape(-1),
        item_emb.reshape(-1), last_move_emb.reshape(-1)]).reshape(-1, SEG)

    bases = []
    off = 0
    for w in TABLE_WIDTHS:
        bases.append(off)
        off += N_ROWS * (w // SEG)
    sp_b, mv_b, ab_b, it_b, lm_b = bases

    cols = []
    for k in range(4):
        cols.append(4 * ids[..., 0] + (sp_b + k))
    for f in range(1, 5):
        for k in range(2):
            cols.append(2 * ids[..., f] + (mv_b + k))
    cols.append(ids[..., 5] + ab_b)
    cols.append(ids[..., 6] + it_b)
    for k in range(2):
        cols.append(2 * ids[..., 7] + (lm_b + k))
    idx16 = jnp.stack(cols, axis=-1)
    idx = idx16.reshape(NCHUNKS, GLEN)

    out = _run(idx, flat)
    return out.reshape(BATCH, N_TOKENS, D_OUT)


# speedup vs baseline: 1.3072x; 1.3072x over previous
import jax
import jax.numpy as jnp
from jax import lax
from jax.experimental import pallas as pl
from jax.experimental.pallas import tpu as pltpu
from jax.experimental.pallas import tpu_sc as plsc

BATCH = 4096
N_TOKENS = 50
TOKENS = BATCH * N_TOKENS
CHUNK = 128
NCHUNKS = TOKENS // CHUNK
NW = 32
CPW = NCHUNKS // NW
D_OUT = 256
N_ROWS = 100000

FIELDS = (
    (0, 0, 64, 0),
    (1, 64, 32, 1),
    (2, 96, 32, 1),
    (3, 128, 32, 1),
    (4, 160, 32, 1),
    (5, 192, 16, 2),
    (6, 208, 16, 3),
    (7, 224, 32, 4),
)
TABLE_WIDTHS = (64, 32, 16, 16, 32)


def _body(idx_hbm, sp_hbm, mv_hbm, ab_hbm, it_hbm, lm_hbm, out_hbm,
          idxv, bufs0, bufs1, gsem0, gsem1, wsem0, wsem1, isem):
    tables = (sp_hbm, mv_hbm, ab_hbm, it_hbm, lm_hbm)
    bufs = (bufs0, bufs1)
    gsems = (gsem0, gsem1)
    wsems = (wsem0, wsem1)
    wid = lax.axis_index("s") * 2 + lax.axis_index("c")
    chunk0 = wid * CPW

    def fire_gathers(b):
        for i, (row, _, _, targ) in enumerate(FIELDS):
            pltpu.async_copy(tables[targ].at[idxv.at[b, row]], bufs[b][i],
                             gsems[b])

    def wait_gathers(b):
        for i, (row, _, _, targ) in enumerate(FIELDS):
            pltpu.make_async_copy(tables[targ].at[idxv.at[b, row]], bufs[b][i],
                                  gsems[b]).wait()

    def fire_writes(b, g):
        tok = pl.multiple_of(g * CHUNK, CHUNK)
        for i, (_, col, w, _) in enumerate(FIELDS):
            pltpu.async_copy(
                bufs[b][i], out_hbm.at[pl.ds(tok, CHUNK), pl.ds(col, w)],
                wsems[b])

    def wait_writes(b):
        for i, (_, col, w, _) in enumerate(FIELDS):
            pltpu.make_async_copy(
                bufs[b][i], out_hbm.at[pl.ds(0, CHUNK), pl.ds(col, w)],
                wsems[b]).wait()

    pltpu.sync_copy(idx_hbm.at[chunk0], idxv.at[0])
    fire_gathers(0)

    def pair(jj, carry):
        for b in (0, 1):
            j = jj * 2 + b
            g = chunk0 + j

            @pl.when(j + 1 < CPW)
            def _prefetch_idx():
                pltpu.async_copy(idx_hbm.at[g + 1], idxv.at[1 - b], isem)

            @pl.when(j >= 1)
            def _():
                wait_writes(1 - b)

            @pl.when(j + 1 < CPW)
            def _next_gathers():
                pltpu.make_async_copy(idx_hbm.at[g + 1], idxv.at[1 - b],
                                      isem).wait()
                fire_gathers(1 - b)

            wait_gathers(b)
            fire_writes(b, g)

        return carry

    lax.fori_loop(0, CPW // 2, pair, 0)
    wait_writes((CPW - 1) % 2)


@jax.jit
def _run(idx, sp, mv, ab, it, lm):
    def field_bufs():
        return tuple(pltpu.VMEM((CHUNK, w), jnp.float32) for _, _, w, _ in FIELDS)
    scratch = [
        pltpu.VMEM((2, 8, CHUNK), jnp.int32),
        field_bufs(),
        field_bufs(),
        pltpu.SemaphoreType.DMA,
        pltpu.SemaphoreType.DMA,
        pltpu.SemaphoreType.DMA,
        pltpu.SemaphoreType.DMA,
        pltpu.SemaphoreType.DMA,
    ]
    kern = pl.kernel(
        _body,
        out_type=jax.ShapeDtypeStruct((TOKENS, D_OUT), jnp.float32),
        mesh=plsc.VectorSubcoreMesh(core_axis_name="c", subcore_axis_name="s"),
        scratch_types=scratch,
        compiler_params=pltpu.CompilerParams(use_tc_tiling_on_sc=False),
    )
    return kern(idx, sp, mv, ab, it, lm)


def kernel(int_ids, species_emb, move_emb, ability_emb, item_emb, last_move_emb):
    ids = int_ids.astype(jnp.int32)
    idx_t = ids.reshape(NCHUNKS, CHUNK, 8).transpose(0, 2, 1)
    pieces = [idx_t.reshape(-1)]
    for t in (species_emb, move_emb, ability_emb, item_emb, last_move_emb):
        pieces.append(lax.bitcast_convert_type(t, jnp.int32).reshape(-1))
    flat = jnp.concatenate(pieces)

    idx = lax.slice(flat, (0,), (NCHUNKS * 8 * CHUNK,)).reshape(
        NCHUNKS, 8, CHUNK)
    off = NCHUNKS * 8 * CHUNK
    views = []
    for w in TABLE_WIDTHS:
        v = lax.slice(flat, (off,), (off + N_ROWS * w,)).reshape(N_ROWS, w)
        views.append(lax.bitcast_convert_type(v, jnp.float32))
        off += N_ROWS * w

    out = _run(idx, *views)
    return out.reshape(BATCH, N_TOKENS, D_OUT)
